# baseline (speedup 1.0000x reference)
import functools

import jax
import jax.numpy as jnp
from jax import lax
from jax.experimental import pallas as pl
from jax.experimental.pallas import tpu as pltpu
from jax.experimental.pallas import tpu_sc as plsc

N = 10000
E = 320000
D = 128
R = 200


def _ffn_body(x_ref, w_ref, b_ref, o_ref):
    y = lax.dot_general(
        x_ref[...], w_ref[...], (((1,), (0,)), ((), ())),
        preferred_element_type=jnp.float32,
        precision=lax.Precision.HIGHEST,
    )
    o_ref[...] = jnp.maximum(y + b_ref[...], 0.0)


def _transform(x, W, b, blk):
    n = x.shape[0]
    assert n % blk == 0
    return pl.pallas_call(
        _ffn_body,
        grid=(n // blk,),
        in_specs=[
            pl.BlockSpec((blk, D), lambda i: (i, 0)),
            pl.BlockSpec((D, D), lambda i: (0, 0)),
            pl.BlockSpec((1, D), lambda i: (0, 0)),
        ],
        out_specs=pl.BlockSpec((blk, D), lambda i: (i, 0)),
        out_shape=jax.ShapeDtypeStruct((n, D), jnp.float32),
    )(x, W, b.reshape(1, D))


_INFO = plsc.get_sparse_core_info()
_NC, _NS, _L = _INFO.num_cores, _INFO.num_subcores, _INFO.num_lanes
_NW = _NC * _NS
_C = 128
_NCH = E // _C
_NCW = _NCH // _NW
_NEXTRA = _NCH - _NCW * _NW
_NPAIR = (_NCW + 1) // 2
_NG = _C // _L


_GTR_DNUMS = lax.GatherDimensionNumbers(
    offset_dims=(), collapsed_slice_dims=(0,), start_index_map=(0,))


def _lane_shuffle(v, perm):
    return lax.gather(v, perm[:, None], _GTR_DNUMS, (1,),
                      mode=lax.GatherScatterMode.PROMISE_IN_BOUNDS)


def _sc_body(zh_hbm, zt_hbm, rr_hbm, gidx_hbm, out_hbm,
             gidx0, gidx1, h0, h1, t0, t1, rtab, outv, sem0, sem1):
    wid = lax.axis_index("s") * _NC + lax.axis_index("c")
    lane = lax.iota(jnp.int32, _L)
    nc = jnp.where(wid < _NEXTRA, _NCW + 1, _NCW)

    pltpu.sync_copy(rr_hbm, rtab)

    gidx_bufs = (gidx0, gidx1)
    h_bufs = (h0, h1)
    t_bufs = (t0, t1)
    sems = (sem0, sem1)

    def issue(b, k):
        ci = wid + _NW * k
        pltpu.sync_copy(gidx_hbm.at[pl.ds(ci * (3 * _C), 3 * _C)],
                        gidx_bufs[b])
        pltpu.async_copy(zh_hbm.at[gidx_bufs[b].at[pl.ds(0, _C)]],
                         h_bufs[b], sems[b])
        pltpu.async_copy(zt_hbm.at[gidx_bufs[b].at[pl.ds(_C, _C)]],
                         t_bufs[b], sems[b])

    def wait(b):
        pltpu.make_async_copy(
            zh_hbm.at[pl.ds(0, _C)], h_bufs[b], sems[b]).wait()
        pltpu.make_async_copy(
            zt_hbm.at[pl.ds(0, _C)], t_bufs[b], sems[b]).wait()

    def compute(b, k):
        hrow, trow, idxb = h_bufs[b], t_bufs[b], gidx_bufs[b]

        @plsc.parallel_loop(0, _NG, unroll=2)
        def group(g):
            vec = jnp.zeros((_L,), jnp.float32)
            rvec = idxb[pl.ds(2 * _C + g * _L, _L)]
            for j in range(_L):
                e = g * _L + j
                r = rvec[j]
                acc0 = jnp.zeros((_L,), jnp.float32)
                acc1 = jnp.zeros((_L,), jnp.float32)
                for d in range(0, D // _L, 2):
                    s0 = pl.ds(d * _L, _L)
                    s1 = pl.ds((d + 1) * _L, _L)
                    acc0 = acc0 + hrow[e, s0] * trow[e, s0] * rtab[r, s0]
                    acc1 = acc1 + hrow[e, s1] * trow[e, s1] * rtab[r, s1]
                acc = acc0 + acc1
                for sh in (8, 4, 2, 1):
                    acc = acc + _lane_shuffle(acc, lax.bitwise_xor(lane, sh))
                vec = jnp.where(lane == j, acc, vec)
            outv[pl.ds(g * _L, _L)] = 1.0 / (1.0 + jnp.exp(-vec))

        ci = wid + _NW * k
        pltpu.sync_copy(outv, out_hbm.at[pl.ds(ci * _C, _C)])

    issue(0, 0)

    def pair(p, carry):
        k0 = 2 * p
        issue(1, k0 + 1)
        wait(0)
        compute(0, k0)
        issue(0, jnp.minimum(k0 + 2, nc - 1))
        wait(1)
        compute(1, k0 + 1)
        return carry

    lax.fori_loop(0, _NPAIR, pair, 0)
    wait(0)

    @pl.when(nc > 2 * _NPAIR)
    def _():
        compute(0, nc - 1)


@functools.partial(
    pl.kernel,
    mesh=plsc.VectorSubcoreMesh(core_axis_name="c", subcore_axis_name="s"),
    compiler_params=pltpu.CompilerParams(needs_layout_passes=False),
    out_type=jax.ShapeDtypeStruct((E,), jnp.float32),
    scratch_types=[
        pltpu.VMEM((3 * _C,), jnp.int32),
        pltpu.VMEM((3 * _C,), jnp.int32),
        pltpu.VMEM((_C, D), jnp.float32),
        pltpu.VMEM((_C, D), jnp.float32),
        pltpu.VMEM((_C, D), jnp.float32),
        pltpu.VMEM((_C, D), jnp.float32),
        pltpu.VMEM((R, D), jnp.float32),
        pltpu.VMEM((_C,), jnp.float32),
        pltpu.SemaphoreType.DMA,
        pltpu.SemaphoreType.DMA,
    ],
)
def _sc_edge_score(zh, zt, rr, gidx, out,
                   gidx0, gidx1, h0, h1, t0, t1, rtab, outv, sem0, sem1):
    _sc_body(zh, zt, rr, gidx, out,
             gidx0, gidx1, h0, h1, t0, t1, rtab, outv, sem0, sem1)


def _pack_gidx(head, tail, rel):
    cols = jnp.stack([head.reshape(_NCH, _C), tail.reshape(_NCH, _C),
                      rel.reshape(_NCH, _C)], axis=1)
    return cols.reshape(-1)


def kernel(z, edge_index, rel_type, emb_rel,
           W_head, b_head, W_tail, b_tail, W_rel, b_rel):
    zh = _transform(z, W_head, b_head, 1000)
    zt = _transform(z, W_tail, b_tail, 1000)
    rr = _transform(emb_rel, W_rel, b_rel, R)
    gidx = _pack_gidx(edge_index[0], edge_index[1], rel_type)
    return _sc_edge_score(zh, zt, rr, gidx)


# speedup vs baseline: 1.4505x; 1.4505x over previous
import functools

import jax
import jax.numpy as jnp
from jax import lax
from jax.experimental import pallas as pl
from jax.experimental.pallas import tpu as pltpu
from jax.experimental.pallas import tpu_sc as plsc

N = 10000
E = 320000
D = 128
R = 200


def _ffn_body(x_ref, w_ref, b_ref, o_ref):
    y = lax.dot_general(
        x_ref[...], w_ref[...], (((1,), (0,)), ((), ())),
        preferred_element_type=jnp.float32,
        precision=lax.Precision.HIGHEST,
    )
    o_ref[...] = jnp.maximum(y + b_ref[...], 0.0)


def _transform(x, W, b, blk):
    n = x.shape[0]
    assert n % blk == 0
    return pl.pallas_call(
        _ffn_body,
        grid=(n // blk,),
        in_specs=[
            pl.BlockSpec((blk, D), lambda i: (i, 0)),
            pl.BlockSpec((D, D), lambda i: (0, 0)),
            pl.BlockSpec((1, D), lambda i: (0, 0)),
        ],
        out_specs=pl.BlockSpec((blk, D), lambda i: (i, 0)),
        out_shape=jax.ShapeDtypeStruct((n, D), jnp.float32),
    )(x, W, b.reshape(1, D))


_INFO = plsc.get_sparse_core_info()
_NC, _NS, _L = _INFO.num_cores, _INFO.num_subcores, _INFO.num_lanes
_NW = _NC * _NS
_C = 128
_NCH = E // _C
_NCW = _NCH // _NW
_NEXTRA = _NCH - _NCW * _NW
_NPAIR = (_NCW + 1) // 2
_NG = _C // _L


_GTR_DNUMS = lax.GatherDimensionNumbers(
    offset_dims=(), collapsed_slice_dims=(0,), start_index_map=(0,))


def _lane_shuffle(v, perm):
    return lax.gather(v, perm[:, None], _GTR_DNUMS, (1,),
                      mode=lax.GatherScatterMode.PROMISE_IN_BOUNDS)


def _sc_body(zh_hbm, zt_hbm, rr_hbm, gidx_hbm, out_hbm,
             gidx0, gidx1, h0, h1, t0, t1, rtab, outv, sem0, sem1):
    wid = lax.axis_index("s") * _NC + lax.axis_index("c")
    lane = lax.iota(jnp.int32, _L)
    nc = jnp.where(wid < _NEXTRA, _NCW + 1, _NCW)

    pltpu.sync_copy(rr_hbm, rtab)

    gidx_bufs = (gidx0, gidx1)
    h_bufs = (h0, h1)
    t_bufs = (t0, t1)
    sems = (sem0, sem1)

    def issue(b, k):
        ci = wid + _NW * k
        pltpu.sync_copy(gidx_hbm.at[pl.ds(ci * (3 * _C), 3 * _C)],
                        gidx_bufs[b])
        pltpu.async_copy(zh_hbm.at[gidx_bufs[b].at[pl.ds(0, _C)]],
                         h_bufs[b], sems[b])
        pltpu.async_copy(zt_hbm.at[gidx_bufs[b].at[pl.ds(_C, _C)]],
                         t_bufs[b], sems[b])

    def wait(b):
        pltpu.make_async_copy(
            zh_hbm.at[pl.ds(0, _C)], h_bufs[b], sems[b]).wait()
        pltpu.make_async_copy(
            zt_hbm.at[pl.ds(0, _C)], t_bufs[b], sems[b]).wait()

    def compute(b, k):
        hrow, trow, idxb = h_bufs[b], t_bufs[b], gidx_bufs[b]

        @plsc.parallel_loop(0, _NG, unroll=1)
        def group(g):
            vec = jnp.zeros((_L,), jnp.float32)
            rvec = idxb[pl.ds(2 * _C + g * _L, _L)]
            for j in range(_L):
                e = g * _L + j
                r = rvec[j]
                acc0 = jnp.zeros((_L,), jnp.float32)
                acc1 = jnp.zeros((_L,), jnp.float32)
                for d in range(0, D // _L, 2):
                    s0 = pl.ds(d * _L, _L)
                    s1 = pl.ds((d + 1) * _L, _L)
                    acc0 = acc0 + hrow[e, s0] * trow[e, s0] * rtab[r, s0]
                    acc1 = acc1 + hrow[e, s1] * trow[e, s1] * rtab[r, s1]
                acc = acc0 + acc1
                for sh in (8, 4, 2, 1):
                    acc = acc + _lane_shuffle(acc, lax.bitwise_xor(lane, sh))
                vec = jnp.where(lane == j, acc, vec)
            outv[pl.ds(g * _L, _L)] = 1.0 / (1.0 + jnp.exp(-vec))

        ci = wid + _NW * k
        pltpu.sync_copy(outv, out_hbm.at[pl.ds(ci * _C, _C)])

    issue(0, 0)

    def pair(p, carry):
        k0 = 2 * p
        issue(1, k0 + 1)
        wait(0)
        compute(0, k0)
        issue(0, jnp.minimum(k0 + 2, nc - 1))
        wait(1)
        compute(1, k0 + 1)
        return carry

    lax.fori_loop(0, _NPAIR, pair, 0)
    wait(0)

    @pl.when(nc > 2 * _NPAIR)
    def _():
        compute(0, nc - 1)


@functools.partial(
    pl.kernel,
    mesh=plsc.VectorSubcoreMesh(core_axis_name="c", subcore_axis_name="s"),
    compiler_params=pltpu.CompilerParams(needs_layout_passes=False),
    out_type=jax.ShapeDtypeStruct((E,), jnp.float32),
    scratch_types=[
        pltpu.VMEM((3 * _C,), jnp.int32),
        pltpu.VMEM((3 * _C,), jnp.int32),
        pltpu.VMEM((_C, D), jnp.float32),
        pltpu.VMEM((_C, D), jnp.float32),
        pltpu.VMEM((_C, D), jnp.float32),
        pltpu.VMEM((_C, D), jnp.float32),
        pltpu.VMEM((R, D), jnp.float32),
        pltpu.VMEM((_C,), jnp.float32),
        pltpu.SemaphoreType.DMA,
        pltpu.SemaphoreType.DMA,
    ],
)
def _sc_edge_score(zh, zt, rr, gidx, out,
                   gidx0, gidx1, h0, h1, t0, t1, rtab, outv, sem0, sem1):
    _sc_body(zh, zt, rr, gidx, out,
             gidx0, gidx1, h0, h1, t0, t1, rtab, outv, sem0, sem1)


def _pack_gidx(head, tail, rel):
    cols = jnp.stack([head.reshape(_NCH, _C), tail.reshape(_NCH, _C),
                      rel.reshape(_NCH, _C)], axis=1)
    return cols.reshape(-1)


def kernel(z, edge_index, rel_type, emb_rel,
           W_head, b_head, W_tail, b_tail, W_rel, b_rel):
    zh = _transform(z, W_head, b_head, 1000)
    zt = _transform(z, W_tail, b_tail, 1000)
    rr = _transform(emb_rel, W_rel, b_rel, R)
    gidx = _pack_gidx(edge_index[0], edge_index[1], rel_type)
    return _sc_edge_score(zh, zt, rr, gidx)


# speedup vs baseline: 1.5149x; 1.0444x over previous
import functools

import jax
import jax.numpy as jnp
from jax import lax
from jax.experimental import pallas as pl
from jax.experimental.pallas import tpu as pltpu
from jax.experimental.pallas import tpu_sc as plsc

N = 10000
E = 320000
D = 128
R = 200


def _ffn_body(x_ref, w_ref, b_ref, o_ref):
    y = lax.dot_general(
        x_ref[...], w_ref[...], (((1,), (0,)), ((), ())),
        preferred_element_type=jnp.float32,
        precision=lax.Precision.HIGHEST,
    )
    o_ref[...] = jnp.maximum(y + b_ref[...], 0.0)


def _transform(x, W, b, blk):
    n = x.shape[0]
    assert n % blk == 0
    return pl.pallas_call(
        _ffn_body,
        grid=(n // blk,),
        in_specs=[
            pl.BlockSpec((blk, D), lambda i: (i, 0)),
            pl.BlockSpec((D, D), lambda i: (0, 0)),
            pl.BlockSpec((1, D), lambda i: (0, 0)),
        ],
        out_specs=pl.BlockSpec((blk, D), lambda i: (i, 0)),
        out_shape=jax.ShapeDtypeStruct((n, D), jnp.float32),
    )(x, W, b.reshape(1, D))


_INFO = plsc.get_sparse_core_info()
_NC, _NS, _L = _INFO.num_cores, _INFO.num_subcores, _INFO.num_lanes
_NW = _NC * _NS
_C = 128
_NCH = E // _C
_NCW = _NCH // _NW
_NEXTRA = _NCH - _NCW * _NW
_NPAIR = (_NCW + 1) // 2
_NG = _C // _L


_GTR_DNUMS = lax.GatherDimensionNumbers(
    offset_dims=(), collapsed_slice_dims=(0,), start_index_map=(0,))


def _lane_shuffle(v, perm):
    return lax.gather(v, perm[:, None], _GTR_DNUMS, (1,),
                      mode=lax.GatherScatterMode.PROMISE_IN_BOUNDS)


def _sc_body(zh_hbm, zt_hbm, rr_hbm, gidx_hbm, out_hbm,
             gidx0, gidx1, h0, h1, t0, t1, rtab, outv, sem0, sem1):
    wid = lax.axis_index("s") * _NC + lax.axis_index("c")
    lane = lax.iota(jnp.int32, _L)
    nc = jnp.where(wid < _NEXTRA, _NCW + 1, _NCW)

    pltpu.sync_copy(rr_hbm, rtab)

    gidx_bufs = (gidx0, gidx1)
    h_bufs = (h0, h1)
    t_bufs = (t0, t1)
    sems = (sem0, sem1)

    def issue(b, k):
        ci = wid + _NW * k
        pltpu.sync_copy(gidx_hbm.at[pl.ds(ci * (3 * _C), 3 * _C)],
                        gidx_bufs[b])
        pltpu.async_copy(zh_hbm.at[gidx_bufs[b].at[pl.ds(0, _C)]],
                         h_bufs[b], sems[b])
        pltpu.async_copy(zt_hbm.at[gidx_bufs[b].at[pl.ds(_C, _C)]],
                         t_bufs[b], sems[b])

    def wait(b):
        pltpu.make_async_copy(
            zh_hbm.at[pl.ds(0, _C)], h_bufs[b], sems[b]).wait()
        pltpu.make_async_copy(
            zt_hbm.at[pl.ds(0, _C)], t_bufs[b], sems[b]).wait()

    def compute(b, k):
        hrow, trow, idxb = h_bufs[b], t_bufs[b], gidx_bufs[b]

        def group(g, carry):
            vec = jnp.zeros((_L,), jnp.float32)
            rvec = idxb[pl.ds(2 * _C + g * _L, _L)]
            for j in range(_L):
                e = g * _L + j
                r = rvec[j]
                acc0 = jnp.zeros((_L,), jnp.float32)
                acc1 = jnp.zeros((_L,), jnp.float32)
                for d in range(0, D // _L, 2):
                    s0 = pl.ds(d * _L, _L)
                    s1 = pl.ds((d + 1) * _L, _L)
                    acc0 = acc0 + hrow[e, s0] * trow[e, s0] * rtab[r, s0]
                    acc1 = acc1 + hrow[e, s1] * trow[e, s1] * rtab[r, s1]
                acc = acc0 + acc1
                for sh in (8, 4, 2, 1):
                    acc = acc + _lane_shuffle(acc, lax.bitwise_xor(lane, sh))
                vec = jnp.where(lane == j, acc, vec)
            outv[pl.ds(g * _L, _L)] = 1.0 / (1.0 + jnp.exp(-vec))
            return carry

        lax.fori_loop(0, _NG, group, 0)
        ci = wid + _NW * k
        pltpu.sync_copy(outv, out_hbm.at[pl.ds(ci * _C, _C)])

    issue(0, 0)

    def pair(p, carry):
        k0 = 2 * p
        issue(1, k0 + 1)
        wait(0)
        compute(0, k0)
        issue(0, jnp.minimum(k0 + 2, nc - 1))
        wait(1)
        compute(1, k0 + 1)
        return carry

    lax.fori_loop(0, _NPAIR, pair, 0)
    wait(0)

    @pl.when(nc > 2 * _NPAIR)
    def _():
        compute(0, nc - 1)


@functools.partial(
    pl.kernel,
    mesh=plsc.VectorSubcoreMesh(core_axis_name="c", subcore_axis_name="s"),
    compiler_params=pltpu.CompilerParams(needs_layout_passes=False),
    out_type=jax.ShapeDtypeStruct((E,), jnp.float32),
    scratch_types=[
        pltpu.VMEM((3 * _C,), jnp.int32),
        pltpu.VMEM((3 * _C,), jnp.int32),
        pltpu.VMEM((_C, D), jnp.float32),
        pltpu.VMEM((_C, D), jnp.float32),
        pltpu.VMEM((_C, D), jnp.float32),
        pltpu.VMEM((_C, D), jnp.float32),
        pltpu.VMEM((R, D), jnp.float32),
        pltpu.VMEM((_C,), jnp.float32),
        pltpu.SemaphoreType.DMA,
        pltpu.SemaphoreType.DMA,
    ],
)
def _sc_edge_score(zh, zt, rr, gidx, out,
                   gidx0, gidx1, h0, h1, t0, t1, rtab, outv, sem0, sem1):
    _sc_body(zh, zt, rr, gidx, out,
             gidx0, gidx1, h0, h1, t0, t1, rtab, outv, sem0, sem1)


def _pack_gidx(head, tail, rel):
    cols = jnp.stack([head.reshape(_NCH, _C), tail.reshape(_NCH, _C),
                      rel.reshape(_NCH, _C)], axis=1)
    return cols.reshape(-1)


def kernel(z, edge_index, rel_type, emb_rel,
           W_head, b_head, W_tail, b_tail, W_rel, b_rel):
    zh = _transform(z, W_head, b_head, 1000)
    zt = _transform(z, W_tail, b_tail, 1000)
    rr = _transform(emb_rel, W_rel, b_rel, R)
    gidx = _pack_gidx(edge_index[0], edge_index[1], rel_type)
    return _sc_edge_score(zh, zt, rr, gidx)


# speedup vs baseline: 2.1537x; 1.4217x over previous
import functools

import jax
import jax.numpy as jnp
from jax import lax
from jax.experimental import pallas as pl
from jax.experimental.pallas import tpu as pltpu
from jax.experimental.pallas import tpu_sc as plsc

N = 10000
E = 320000
D = 128
R = 200


def _ffn_body(x_ref, w_ref, b_ref, o_ref):
    y = lax.dot_general(
        x_ref[...], w_ref[...], (((1,), (0,)), ((), ())),
        preferred_element_type=jnp.float32,
        precision=lax.Precision.HIGHEST,
    )
    o_ref[...] = jnp.maximum(y + b_ref[...], 0.0)


def _transform(x, W, b, blk):
    n = x.shape[0]
    assert n % blk == 0
    return pl.pallas_call(
        _ffn_body,
        grid=(n // blk,),
        in_specs=[
            pl.BlockSpec((blk, D), lambda i: (i, 0)),
            pl.BlockSpec((D, D), lambda i: (0, 0)),
            pl.BlockSpec((1, D), lambda i: (0, 0)),
        ],
        out_specs=pl.BlockSpec((blk, D), lambda i: (i, 0)),
        out_shape=jax.ShapeDtypeStruct((n, D), jnp.float32),
    )(x, W, b.reshape(1, D))


_INFO = plsc.get_sparse_core_info()
_NC, _NS, _L = _INFO.num_cores, _INFO.num_subcores, _INFO.num_lanes
_NW = _NC * _NS
_C = 128
_NCH = E // _C
_NCW = _NCH // _NW
_NEXTRA = _NCH - _NCW * _NW
_NPAIR = (_NCW + 1) // 2
_NG = _C // _L


_GTR_DNUMS = lax.GatherDimensionNumbers(
    offset_dims=(), collapsed_slice_dims=(0,), start_index_map=(0,))


def _lane_shuffle(v, perm):
    return lax.gather(v, perm[:, None], _GTR_DNUMS, (1,),
                      mode=lax.GatherScatterMode.PROMISE_IN_BOUNDS)


def _sc_body(zh_hbm, zt_hbm, rr_hbm, gidx_hbm, out_hbm,
             gidx0, gidx1, h0, h1, t0, t1, rtab, outv, sem0, sem1):
    wid = lax.axis_index("s") * _NC + lax.axis_index("c")
    lane = lax.iota(jnp.int32, _L)
    nc = jnp.where(wid < _NEXTRA, _NCW + 1, _NCW)

    pltpu.sync_copy(rr_hbm, rtab)

    gidx_bufs = (gidx0, gidx1)
    h_bufs = (h0, h1)
    t_bufs = (t0, t1)
    sems = (sem0, sem1)

    def issue(b, k):
        ci = wid + _NW * k
        pltpu.sync_copy(gidx_hbm.at[pl.ds(ci * (3 * _C), 3 * _C)],
                        gidx_bufs[b])
        pltpu.async_copy(zh_hbm.at[gidx_bufs[b].at[pl.ds(0, _C)]],
                         h_bufs[b], sems[b])
        pltpu.async_copy(zt_hbm.at[gidx_bufs[b].at[pl.ds(_C, _C)]],
                         t_bufs[b], sems[b])

    def wait(b):
        pltpu.make_async_copy(
            zh_hbm.at[pl.ds(0, _C)], h_bufs[b], sems[b]).wait()
        pltpu.make_async_copy(
            zt_hbm.at[pl.ds(0, _C)], t_bufs[b], sems[b]).wait()

    def compute(b, k):
        hrow, trow, idxb = h_bufs[b], t_bufs[b], gidx_bufs[b]

        def group(g, carry):
            vec = jnp.zeros((_L,), jnp.float32)
            rvec = idxb[pl.ds(2 * _C + g * _L, _L)]
            for j in range(_L):
                e = g * _L + j
                r = rvec[j]
                acc = jnp.zeros((_L,), jnp.float32)
                for d in range(D // _L):
                    s = pl.ds(d * _L, _L)
                    acc = acc + hrow[e, s] * trow[e, s] * rtab[r, s]
                for sh in (8, 4, 2, 1):
                    acc = acc + _lane_shuffle(acc, lax.bitwise_xor(lane, sh))
                vec = jnp.where(lane == j, acc, vec)
            outv[pl.ds(g * _L, _L)] = 1.0 / (1.0 + jnp.exp(-vec))
            return carry

        lax.fori_loop(0, _NG, group, 0)
        ci = wid + _NW * k
        pltpu.sync_copy(outv, out_hbm.at[pl.ds(ci * _C, _C)])

    issue(0, 0)

    def pair(p, carry):
        k0 = 2 * p
        issue(1, k0 + 1)
        wait(0)
        compute(0, k0)
        issue(0, jnp.minimum(k0 + 2, nc - 1))
        wait(1)
        compute(1, k0 + 1)
        return carry

    lax.fori_loop(0, _NPAIR, pair, 0)
    wait(0)

    @pl.when(nc > 2 * _NPAIR)
    def _():
        compute(0, nc - 1)


@functools.partial(
    pl.kernel,
    mesh=plsc.VectorSubcoreMesh(core_axis_name="c", subcore_axis_name="s"),
    compiler_params=pltpu.CompilerParams(needs_layout_passes=False),
    out_type=jax.ShapeDtypeStruct((E,), jnp.float32),
    scratch_types=[
        pltpu.VMEM((3 * _C,), jnp.int32),
        pltpu.VMEM((3 * _C,), jnp.int32),
        pltpu.VMEM((_C, D), jnp.float32),
        pltpu.VMEM((_C, D), jnp.float32),
        pltpu.VMEM((_C, D), jnp.float32),
        pltpu.VMEM((_C, D), jnp.float32),
        pltpu.VMEM((R, D), jnp.float32),
        pltpu.VMEM((_C,), jnp.float32),
        pltpu.SemaphoreType.DMA,
        pltpu.SemaphoreType.DMA,
    ],
)
def _sc_edge_score(zh, zt, rr, gidx, out,
                   gidx0, gidx1, h0, h1, t0, t1, rtab, outv, sem0, sem1):
    _sc_body(zh, zt, rr, gidx, out,
             gidx0, gidx1, h0, h1, t0, t1, rtab, outv, sem0, sem1)


def _pack_gidx(head, tail, rel):
    cols = jnp.stack([head.reshape(_NCH, _C), tail.reshape(_NCH, _C),
                      rel.reshape(_NCH, _C)], axis=1)
    return cols.reshape(-1)


def kernel(z, edge_index, rel_type, emb_rel,
           W_head, b_head, W_tail, b_tail, W_rel, b_rel):
    zh = _transform(z, W_head, b_head, 1000)
    zt = _transform(z, W_tail, b_tail, 1000)
    rr = _transform(emb_rel, W_rel, b_rel, R)
    gidx = _pack_gidx(edge_index[0], edge_index[1], rel_type)
    return _sc_edge_score(zh, zt, rr, gidx)


# speedup vs baseline: 3.2534x; 1.5106x over previous
import functools

import jax
import jax.numpy as jnp
from jax import lax
from jax.experimental import pallas as pl
from jax.experimental.pallas import tpu as pltpu
from jax.experimental.pallas import tpu_sc as plsc

N = 10000
E = 320000
D = 128
R = 200


def _ffn_body(x_ref, w_ref, b_ref, o_ref):
    y = lax.dot_general(
        x_ref[...], w_ref[...], (((1,), (0,)), ((), ())),
        preferred_element_type=jnp.float32,
        precision=lax.Precision.HIGHEST,
    )
    o_ref[...] = jnp.maximum(y + b_ref[...], 0.0)


def _transform(x, W, b, blk):
    n = x.shape[0]
    assert n % blk == 0
    return pl.pallas_call(
        _ffn_body,
        grid=(n // blk,),
        in_specs=[
            pl.BlockSpec((blk, D), lambda i: (i, 0)),
            pl.BlockSpec((D, D), lambda i: (0, 0)),
            pl.BlockSpec((1, D), lambda i: (0, 0)),
        ],
        out_specs=pl.BlockSpec((blk, D), lambda i: (i, 0)),
        out_shape=jax.ShapeDtypeStruct((n, D), jnp.float32),
    )(x, W, b.reshape(1, D))


_INFO = plsc.get_sparse_core_info()
_NC, _NS, _L = _INFO.num_cores, _INFO.num_subcores, _INFO.num_lanes
_NW = _NC * _NS
_C = 128
_NCH = E // _C
_NCW = _NCH // _NW
_NEXTRA = _NCH - _NCW * _NW
_NPAIR = (_NCW + 1) // 2
_NG = _C // _L


_GTR_DNUMS = lax.GatherDimensionNumbers(
    offset_dims=(), collapsed_slice_dims=(0,), start_index_map=(0,))


def _lane_shuffle(v, perm):
    return lax.gather(v, perm[:, None], _GTR_DNUMS, (1,),
                      mode=lax.GatherScatterMode.PROMISE_IN_BOUNDS)


def _sc_body(zh_hbm, zt_hbm, rr_hbm, gidx_hbm, out_hbm,
             gidx0, gidx1, h0, h1, t0, t1, rtab, outv, scr, sem0, sem1):
    wid = lax.axis_index("s") * _NC + lax.axis_index("c")
    lane = lax.iota(jnp.int32, _L)
    nc = jnp.where(wid < _NEXTRA, _NCW + 1, _NCW)

    pltpu.sync_copy(rr_hbm, rtab)

    gidx_bufs = (gidx0, gidx1)
    h_bufs = (h0, h1)
    t_bufs = (t0, t1)
    sems = (sem0, sem1)

    def issue(b, k):
        ci = wid + _NW * k
        pltpu.sync_copy(gidx_hbm.at[pl.ds(ci * (3 * _C), 3 * _C)],
                        gidx_bufs[b])
        pltpu.async_copy(zh_hbm.at[gidx_bufs[b].at[pl.ds(0, _C)]],
                         h_bufs[b], sems[b])
        pltpu.async_copy(zt_hbm.at[gidx_bufs[b].at[pl.ds(_C, _C)]],
                         t_bufs[b], sems[b])

    def wait(b):
        pltpu.make_async_copy(
            zh_hbm.at[pl.ds(0, _C)], h_bufs[b], sems[b]).wait()
        pltpu.make_async_copy(
            zt_hbm.at[pl.ds(0, _C)], t_bufs[b], sems[b]).wait()

    def compute(b, k):
        hrow, trow, idxb = h_bufs[b], t_bufs[b], gidx_bufs[b]
        lane17 = lane * 17

        def group(g, carry):
            rvec = idxb[pl.ds(2 * _C + g * _L, _L)]
            for j in range(_L):
                e = g * _L + j
                r = rvec[j]
                acc = jnp.zeros((_L,), jnp.float32)
                for d in range(D // _L):
                    s = pl.ds(d * _L, _L)
                    acc = acc + hrow[e, s] * trow[e, s] * rtab[r, s]
                plsc.store_scatter(scr, [lane17 + j], acc)
            vec = jnp.zeros((_L,), jnp.float32)
            for i in range(_L):
                vec = vec + plsc.load_gather(scr, [lane + i * 17])
            outv[pl.ds(g * _L, _L)] = 1.0 / (1.0 + jnp.exp(-vec))
            return carry

        lax.fori_loop(0, _NG, group, 0)
        ci = wid + _NW * k
        pltpu.sync_copy(outv, out_hbm.at[pl.ds(ci * _C, _C)])

    issue(0, 0)

    def pair(p, carry):
        k0 = 2 * p
        issue(1, k0 + 1)
        wait(0)
        compute(0, k0)
        issue(0, jnp.minimum(k0 + 2, nc - 1))
        wait(1)
        compute(1, k0 + 1)
        return carry

    lax.fori_loop(0, _NPAIR, pair, 0)
    wait(0)

    @pl.when(nc > 2 * _NPAIR)
    def _():
        compute(0, nc - 1)


@functools.partial(
    pl.kernel,
    mesh=plsc.VectorSubcoreMesh(core_axis_name="c", subcore_axis_name="s"),
    compiler_params=pltpu.CompilerParams(needs_layout_passes=False),
    out_type=jax.ShapeDtypeStruct((E,), jnp.float32),
    scratch_types=[
        pltpu.VMEM((3 * _C,), jnp.int32),
        pltpu.VMEM((3 * _C,), jnp.int32),
        pltpu.VMEM((_C, D), jnp.float32),
        pltpu.VMEM((_C, D), jnp.float32),
        pltpu.VMEM((_C, D), jnp.float32),
        pltpu.VMEM((_C, D), jnp.float32),
        pltpu.VMEM((R, D), jnp.float32),
        pltpu.VMEM((_C,), jnp.float32),
        pltpu.VMEM((_L * 17,), jnp.float32),
        pltpu.SemaphoreType.DMA,
        pltpu.SemaphoreType.DMA,
    ],
)
def _sc_edge_score(zh, zt, rr, gidx, out,
                   gidx0, gidx1, h0, h1, t0, t1, rtab, outv, scr,
                   sem0, sem1):
    _sc_body(zh, zt, rr, gidx, out,
             gidx0, gidx1, h0, h1, t0, t1, rtab, outv, scr, sem0, sem1)


def _pack_gidx(head, tail, rel):
    cols = jnp.stack([head.reshape(_NCH, _C), tail.reshape(_NCH, _C),
                      rel.reshape(_NCH, _C)], axis=1)
    return cols.reshape(-1)


def kernel(z, edge_index, rel_type, emb_rel,
           W_head, b_head, W_tail, b_tail, W_rel, b_rel):
    zh = _transform(z, W_head, b_head, 1000)
    zt = _transform(z, W_tail, b_tail, 1000)
    rr = _transform(emb_rel, W_rel, b_rel, R)
    gidx = _pack_gidx(edge_index[0], edge_index[1], rel_type)
    return _sc_edge_score(zh, zt, rr, gidx)


# speedup vs baseline: 3.2684x; 1.0046x over previous
import functools

import jax
import jax.numpy as jnp
from jax import lax
from jax.experimental import pallas as pl
from jax.experimental.pallas import tpu as pltpu
from jax.experimental.pallas import tpu_sc as plsc

N = 10000
E = 320000
D = 128
R = 200


def _mm(x, w):
    return lax.dot_general(
        x, w, (((1,), (0,)), ((), ())),
        preferred_element_type=jnp.float32,
        precision=lax.Precision.HIGHEST,
    )


def _ffn2_body(x_ref, wh_ref, bh_ref, wt_ref, bt_ref, oh_ref, ot_ref):
    x = x_ref[...]
    oh_ref[...] = jnp.maximum(_mm(x, wh_ref[...]) + bh_ref[...], 0.0)
    ot_ref[...] = jnp.maximum(_mm(x, wt_ref[...]) + bt_ref[...], 0.0)


def _transform2(x, Wh, bh, Wt, bt, blk):
    n = x.shape[0]
    assert n % blk == 0
    full = pl.BlockSpec((D, D), lambda i: (0, 0))
    bias = pl.BlockSpec((1, D), lambda i: (0, 0))
    rows = pl.BlockSpec((blk, D), lambda i: (i, 0))
    return pl.pallas_call(
        _ffn2_body,
        grid=(n // blk,),
        in_specs=[rows, full, bias, full, bias],
        out_specs=(rows, rows),
        out_shape=(jax.ShapeDtypeStruct((n, D), jnp.float32),
                   jax.ShapeDtypeStruct((n, D), jnp.float32)),
    )(x, Wh, bh.reshape(1, D), Wt, bt.reshape(1, D))


def _ffn_body(x_ref, w_ref, b_ref, o_ref):
    o_ref[...] = jnp.maximum(_mm(x_ref[...], w_ref[...]) + b_ref[...], 0.0)


def _transform(x, W, b, blk):
    n = x.shape[0]
    assert n % blk == 0
    return pl.pallas_call(
        _ffn_body,
        grid=(n // blk,),
        in_specs=[
            pl.BlockSpec((blk, D), lambda i: (i, 0)),
            pl.BlockSpec((D, D), lambda i: (0, 0)),
            pl.BlockSpec((1, D), lambda i: (0, 0)),
        ],
        out_specs=pl.BlockSpec((blk, D), lambda i: (i, 0)),
        out_shape=jax.ShapeDtypeStruct((n, D), jnp.float32),
    )(x, W, b.reshape(1, D))


_INFO = plsc.get_sparse_core_info()
_NC, _NS, _L = _INFO.num_cores, _INFO.num_subcores, _INFO.num_lanes
_NW = _NC * _NS
_C = 160
_NCH = E // _C
_NCW = _NCH // _NW
_NEXTRA = _NCH - _NCW * _NW
_NPAIR = (_NCW + 1) // 2
_NG = _C // _L


_GTR_DNUMS = lax.GatherDimensionNumbers(
    offset_dims=(), collapsed_slice_dims=(0,), start_index_map=(0,))


def _lane_shuffle(v, perm):
    return lax.gather(v, perm[:, None], _GTR_DNUMS, (1,),
                      mode=lax.GatherScatterMode.PROMISE_IN_BOUNDS)


def _sc_body(zh_hbm, zt_hbm, rr_hbm, gidx_hbm, out_hbm,
             gidx0, gidx1, h0, h1, t0, t1, rtab, outv, scr, sem0, sem1):
    wid = lax.axis_index("s") * _NC + lax.axis_index("c")
    lane = lax.iota(jnp.int32, _L)
    nc = jnp.where(wid < _NEXTRA, _NCW + 1, _NCW)

    pltpu.sync_copy(rr_hbm, rtab)

    gidx_bufs = (gidx0, gidx1)
    h_bufs = (h0, h1)
    t_bufs = (t0, t1)
    sems = (sem0, sem1)

    def issue(b, k):
        ci = wid + _NW * k
        pltpu.sync_copy(gidx_hbm.at[pl.ds(ci * (3 * _C), 3 * _C)],
                        gidx_bufs[b])
        pltpu.async_copy(zh_hbm.at[gidx_bufs[b].at[pl.ds(0, _C)]],
                         h_bufs[b], sems[b])
        pltpu.async_copy(zt_hbm.at[gidx_bufs[b].at[pl.ds(_C, _C)]],
                         t_bufs[b], sems[b])

    def wait(b):
        pltpu.make_async_copy(
            zh_hbm.at[pl.ds(0, _C)], h_bufs[b], sems[b]).wait()
        pltpu.make_async_copy(
            zt_hbm.at[pl.ds(0, _C)], t_bufs[b], sems[b]).wait()

    def compute(b, k):
        hrow, trow, idxb = h_bufs[b], t_bufs[b], gidx_bufs[b]
        lane17 = lane * 17

        def group(g, carry):
            rvec = idxb[pl.ds(2 * _C + g * _L, _L)]
            for j in range(_L):
                e = g * _L + j
                r = rvec[j]
                acc = jnp.zeros((_L,), jnp.float32)
                for d in range(D // _L):
                    s = pl.ds(d * _L, _L)
                    acc = acc + hrow[e, s] * trow[e, s] * rtab[r, s]
                plsc.store_scatter(scr, [lane17 + j], acc)
            vec = jnp.zeros((_L,), jnp.float32)
            for i in range(_L):
                vec = vec + plsc.load_gather(scr, [lane + i * 17])
            outv[pl.ds(g * _L, _L)] = 1.0 / (1.0 + jnp.exp(-vec))
            return carry

        lax.fori_loop(0, _NG, group, 0)
        ci = wid + _NW * k
        pltpu.sync_copy(outv, out_hbm.at[pl.ds(ci * _C, _C)])

    issue(0, 0)

    def pair(p, carry):
        k0 = 2 * p
        issue(1, k0 + 1)
        wait(0)
        compute(0, k0)
        issue(0, jnp.minimum(k0 + 2, nc - 1))
        wait(1)
        compute(1, k0 + 1)
        return carry

    lax.fori_loop(0, _NPAIR, pair, 0)
    wait(0)

    @pl.when(nc > 2 * _NPAIR)
    def _():
        compute(0, nc - 1)


@functools.partial(
    pl.kernel,
    mesh=plsc.VectorSubcoreMesh(core_axis_name="c", subcore_axis_name="s"),
    compiler_params=pltpu.CompilerParams(needs_layout_passes=False),
    out_type=jax.ShapeDtypeStruct((E,), jnp.float32),
    scratch_types=[
        pltpu.VMEM((3 * _C,), jnp.int32),
        pltpu.VMEM((3 * _C,), jnp.int32),
        pltpu.VMEM((_C, D), jnp.float32),
        pltpu.VMEM((_C, D), jnp.float32),
        pltpu.VMEM((_C, D), jnp.float32),
        pltpu.VMEM((_C, D), jnp.float32),
        pltpu.VMEM((R, D), jnp.float32),
        pltpu.VMEM((_C,), jnp.float32),
        pltpu.VMEM((_L * 17,), jnp.float32),
        pltpu.SemaphoreType.DMA,
        pltpu.SemaphoreType.DMA,
    ],
)
def _sc_edge_score(zh, zt, rr, gidx, out,
                   gidx0, gidx1, h0, h1, t0, t1, rtab, outv, scr,
                   sem0, sem1):
    _sc_body(zh, zt, rr, gidx, out,
             gidx0, gidx1, h0, h1, t0, t1, rtab, outv, scr, sem0, sem1)


def _pack_gidx(head, tail, rel):
    cols = jnp.stack([head.reshape(_NCH, _C), tail.reshape(_NCH, _C),
                      rel.reshape(_NCH, _C)], axis=1)
    return cols.reshape(-1)


def kernel(z, edge_index, rel_type, emb_rel,
           W_head, b_head, W_tail, b_tail, W_rel, b_rel):
    zh, zt = _transform2(z, W_head, b_head, W_tail, b_tail, 2000)
    rr = _transform(emb_rel, W_rel, b_rel, R)
    gidx = _pack_gidx(edge_index[0], edge_index[1], rel_type)
    return _sc_edge_score(zh, zt, rr, gidx)


# speedup vs baseline: 3.7016x; 1.1326x over previous
import functools

import jax
import jax.numpy as jnp
from jax import lax
from jax.experimental import pallas as pl
from jax.experimental.pallas import tpu as pltpu
from jax.experimental.pallas import tpu_sc as plsc

N = 10000
E = 320000
D = 128
R = 200


def _mm(x, w):
    return lax.dot_general(
        x, w, (((1,), (0,)), ((), ())),
        preferred_element_type=jnp.float32,
        precision=lax.Precision.HIGHEST,
    )


def _ffn2_body(x_ref, wh_ref, bh_ref, wt_ref, bt_ref, oh_ref, ot_ref):
    x = x_ref[...]
    oh_ref[...] = jnp.maximum(_mm(x, wh_ref[...]) + bh_ref[...], 0.0)
    ot_ref[...] = jnp.maximum(_mm(x, wt_ref[...]) + bt_ref[...], 0.0)


def _transform2(x, Wh, bh, Wt, bt, blk):
    n = x.shape[0]
    assert n % blk == 0
    full = pl.BlockSpec((D, D), lambda i: (0, 0))
    bias = pl.BlockSpec((1, D), lambda i: (0, 0))
    rows = pl.BlockSpec((blk, D), lambda i: (i, 0))
    return pl.pallas_call(
        _ffn2_body,
        grid=(n // blk,),
        in_specs=[rows, full, bias, full, bias],
        out_specs=(rows, rows),
        out_shape=(jax.ShapeDtypeStruct((n, D), jnp.float32),
                   jax.ShapeDtypeStruct((n, D), jnp.float32)),
    )(x, Wh, bh.reshape(1, D), Wt, bt.reshape(1, D))


def _ffn_body(x_ref, w_ref, b_ref, o_ref):
    o_ref[...] = jnp.maximum(_mm(x_ref[...], w_ref[...]) + b_ref[...], 0.0)


def _transform(x, W, b, blk):
    n = x.shape[0]
    assert n % blk == 0
    return pl.pallas_call(
        _ffn_body,
        grid=(n // blk,),
        in_specs=[
            pl.BlockSpec((blk, D), lambda i: (i, 0)),
            pl.BlockSpec((D, D), lambda i: (0, 0)),
            pl.BlockSpec((1, D), lambda i: (0, 0)),
        ],
        out_specs=pl.BlockSpec((blk, D), lambda i: (i, 0)),
        out_shape=jax.ShapeDtypeStruct((n, D), jnp.float32),
    )(x, W, b.reshape(1, D))


_INFO = plsc.get_sparse_core_info()
_NC, _NS, _L = _INFO.num_cores, _INFO.num_subcores, _INFO.num_lanes
_NW = _NC * _NS
_C = 160
_NCH = E // _C
_NCW = _NCH // _NW
_NEXTRA = _NCH - _NCW * _NW
_NPAIR = (_NCW + 1) // 2
_NCP = _NCW + 1
_NG = _C // _L


_GTR_DNUMS = lax.GatherDimensionNumbers(
    offset_dims=(), collapsed_slice_dims=(0,), start_index_map=(0,))


def _lane_shuffle(v, perm):
    return lax.gather(v, perm[:, None], _GTR_DNUMS, (1,),
                      mode=lax.GatherScatterMode.PROMISE_IN_BOUNDS)


def _sc_body(zh_hbm, zt_hbm, rr_hbm, gidx_hbm, relp_hbm, out_hbm,
             gidx0, gidx1, relv, h0, h1, t0, t1, rtab, outv0, outv1, scr,
             sem0, sem1, semi0, semi1, semo0, semo1):
    wid = lax.axis_index("s") * _NC + lax.axis_index("c")
    lane = lax.iota(jnp.int32, _L)
    nc = jnp.where(wid < _NEXTRA, _NCW + 1, _NCW)

    pltpu.sync_copy(rr_hbm, rtab)
    pltpu.sync_copy(relp_hbm.at[pl.ds(wid * (_NCP * _C), _NCP * _C)], relv)

    gidx_bufs = (gidx0, gidx1)
    h_bufs = (h0, h1)
    t_bufs = (t0, t1)
    out_bufs = (outv0, outv1)
    sems = (sem0, sem1)
    semis = (semi0, semi1)
    semos = (semo0, semo1)

    def stage_idx(b, k):
        ci = wid + _NW * jnp.minimum(k, nc - 1)
        pltpu.async_copy(gidx_hbm.at[pl.ds(ci * (2 * _C), 2 * _C)],
                         gidx_bufs[b], semis[b])

    def wait_idx(b):
        pltpu.make_async_copy(
            gidx_hbm.at[pl.ds(0, 2 * _C)], gidx_bufs[b], semis[b]).wait()

    def gathers(b):
        pltpu.async_copy(zh_hbm.at[gidx_bufs[b].at[pl.ds(0, _C)]],
                         h_bufs[b], sems[b])
        pltpu.async_copy(zt_hbm.at[gidx_bufs[b].at[pl.ds(_C, _C)]],
                         t_bufs[b], sems[b])

    def wait_g(b):
        pltpu.make_async_copy(
            zh_hbm.at[pl.ds(0, _C)], h_bufs[b], sems[b]).wait()
        pltpu.make_async_copy(
            zt_hbm.at[pl.ds(0, _C)], t_bufs[b], sems[b]).wait()

    def wait_out(b):
        pltpu.make_async_copy(
            out_bufs[b], out_hbm.at[pl.ds(0, _C)], semos[b]).wait()

    def compute(b, k):
        hrow, trow, outv = h_bufs[b], t_bufs[b], out_bufs[b]
        lane17 = lane * 17

        def group(g, carry):
            rvec = relv[pl.ds(k * _C + g * _L, _L)]
            for j in range(_L):
                e = g * _L + j
                r = rvec[j]
                acc = jnp.zeros((_L,), jnp.float32)
                for d in range(D // _L):
                    s = pl.ds(d * _L, _L)
                    acc = acc + hrow[e, s] * trow[e, s] * rtab[r, s]
                plsc.store_scatter(scr, [lane17 + j], acc)
            vec = jnp.zeros((_L,), jnp.float32)
            for i in range(_L):
                vec = vec + plsc.load_gather(scr, [lane + i * 17])
            outv[pl.ds(g * _L, _L)] = 1.0 / (1.0 + jnp.exp(-vec))
            return carry

        lax.fori_loop(0, _NG, group, 0)
        ci = wid + _NW * k
        pltpu.async_copy(outv, out_hbm.at[pl.ds(ci * _C, _C)], semos[b])

    def half(b, k):
        wait_g(b)
        stage_idx(b, k + 2)

        @pl.when(k >= 2)
        def _():
            wait_out(b)
        compute(b, k)
        wait_idx(b)
        gathers(b)

    stage_idx(0, 0)
    stage_idx(1, 1)
    wait_idx(0)
    gathers(0)
    wait_idx(1)
    gathers(1)

    def pair(p, carry):
        half(0, 2 * p)
        half(1, 2 * p + 1)
        return carry

    lax.fori_loop(0, _NPAIR, pair, 0)
    wait_g(0)
    wait_g(1)

    @pl.when(nc > 2 * _NPAIR)
    def _():
        wait_out(0)
        compute(0, nc - 1)
    wait_out(0)
    wait_out(1)


@functools.partial(
    pl.kernel,
    mesh=plsc.VectorSubcoreMesh(core_axis_name="c", subcore_axis_name="s"),
    compiler_params=pltpu.CompilerParams(needs_layout_passes=False),
    out_type=jax.ShapeDtypeStruct((E,), jnp.float32),
    scratch_types=[
        pltpu.VMEM((2 * _C,), jnp.int32),
        pltpu.VMEM((2 * _C,), jnp.int32),
        pltpu.VMEM((_NCP * _C,), jnp.int32),
        pltpu.VMEM((_C, D), jnp.float32),
        pltpu.VMEM((_C, D), jnp.float32),
        pltpu.VMEM((_C, D), jnp.float32),
        pltpu.VMEM((_C, D), jnp.float32),
        pltpu.VMEM((R, D), jnp.float32),
        pltpu.VMEM((_C,), jnp.float32),
        pltpu.VMEM((_C,), jnp.float32),
        pltpu.VMEM((_L * 17,), jnp.float32),
        pltpu.SemaphoreType.DMA,
        pltpu.SemaphoreType.DMA,
        pltpu.SemaphoreType.DMA,
        pltpu.SemaphoreType.DMA,
        pltpu.SemaphoreType.DMA,
        pltpu.SemaphoreType.DMA,
    ],
)
def _sc_edge_score(zh, zt, rr, gidx, relp, out,
                   gidx0, gidx1, relv, h0, h1, t0, t1, rtab, outv0, outv1,
                   scr, sem0, sem1, semi0, semi1, semo0, semo1):
    _sc_body(zh, zt, rr, gidx, relp, out,
             gidx0, gidx1, relv, h0, h1, t0, t1, rtab, outv0, outv1, scr,
             sem0, sem1, semi0, semi1, semo0, semo1)


def _pack_gidx(head, tail):
    cols = jnp.stack([head.reshape(_NCH, _C), tail.reshape(_NCH, _C)],
                     axis=1)
    return cols.reshape(-1)


def _pack_rel(rel):
    rows = rel.reshape(_NCH, _C)
    ids = (jnp.arange(_NW)[:, None]
           + _NW * jnp.arange(_NCP)[None, :])
    return rows[jnp.clip(ids, 0, _NCH - 1)].reshape(-1)


def kernel(z, edge_index, rel_type, emb_rel,
           W_head, b_head, W_tail, b_tail, W_rel, b_rel):
    zh, zt = _transform2(z, W_head, b_head, W_tail, b_tail, 2000)
    rr = _transform(emb_rel, W_rel, b_rel, R)
    gidx = _pack_gidx(edge_index[0], edge_index[1])
    relp = _pack_rel(rel_type)
    return _sc_edge_score(zh, zt, rr, gidx, relp)


# speedup vs baseline: 3.7046x; 1.0008x over previous
import functools

import jax
import jax.numpy as jnp
from jax import lax
from jax.experimental import pallas as pl
from jax.experimental.pallas import tpu as pltpu
from jax.experimental.pallas import tpu_sc as plsc

N = 10000
E = 320000
D = 128
R = 200


def _mm(x, w):
    return lax.dot_general(
        x, w, (((1,), (0,)), ((), ())),
        preferred_element_type=jnp.float32,
        precision=lax.Precision.HIGHEST,
    )


def _ffn2_body(x_ref, wh_ref, bh_ref, wt_ref, bt_ref, oh_ref, ot_ref):
    x = x_ref[...]
    oh_ref[...] = jnp.maximum(_mm(x, wh_ref[...]) + bh_ref[...], 0.0)
    ot_ref[...] = jnp.maximum(_mm(x, wt_ref[...]) + bt_ref[...], 0.0)


def _transform2(x, Wh, bh, Wt, bt, blk):
    n = x.shape[0]
    assert n % blk == 0
    full = pl.BlockSpec((D, D), lambda i: (0, 0))
    bias = pl.BlockSpec((1, D), lambda i: (0, 0))
    rows = pl.BlockSpec((blk, D), lambda i: (i, 0))
    return pl.pallas_call(
        _ffn2_body,
        grid=(n // blk,),
        in_specs=[rows, full, bias, full, bias],
        out_specs=(rows, rows),
        out_shape=(jax.ShapeDtypeStruct((n, D), jnp.float32),
                   jax.ShapeDtypeStruct((n, D), jnp.float32)),
    )(x, Wh, bh.reshape(1, D), Wt, bt.reshape(1, D))


def _ffn_body(x_ref, w_ref, b_ref, o_ref):
    o_ref[...] = jnp.maximum(_mm(x_ref[...], w_ref[...]) + b_ref[...], 0.0)


def _transform(x, W, b, blk):
    n = x.shape[0]
    assert n % blk == 0
    return pl.pallas_call(
        _ffn_body,
        grid=(n // blk,),
        in_specs=[
            pl.BlockSpec((blk, D), lambda i: (i, 0)),
            pl.BlockSpec((D, D), lambda i: (0, 0)),
            pl.BlockSpec((1, D), lambda i: (0, 0)),
        ],
        out_specs=pl.BlockSpec((blk, D), lambda i: (i, 0)),
        out_shape=jax.ShapeDtypeStruct((n, D), jnp.float32),
    )(x, W, b.reshape(1, D))


_INFO = plsc.get_sparse_core_info()
_NC, _NS, _L = _INFO.num_cores, _INFO.num_subcores, _INFO.num_lanes
_NW = _NC * _NS
_C = 160
_NCH = E // _C
_NCW = _NCH // _NW
_NEXTRA = _NCH - _NCW * _NW
_NPAIR = (_NCW + 1) // 2
_NCP = _NCW + 1
_NG = _C // _L


def _sc_body(zh_hbm, zt_hbm, rr_hbm, gidx_hbm, relp_hbm, out_hbm,
             gidx0, gidx1, relv, h0, h1, t0, t1, rtab, outv0, outv1, scr,
             sem0, sem1, semi0, semi1, semo0, semo1):
    wid = lax.axis_index("s") * _NC + lax.axis_index("c")
    lane = lax.iota(jnp.int32, _L)
    nc = jnp.where(wid < _NEXTRA, _NCW + 1, _NCW)

    pltpu.sync_copy(rr_hbm, rtab)
    pltpu.sync_copy(relp_hbm.at[pl.ds(wid * (_NCP * _C), _NCP * _C)], relv)

    gidx_bufs = (gidx0, gidx1)
    h_bufs = (h0, h1)
    t_bufs = (t0, t1)
    out_bufs = (outv0, outv1)
    sems = (sem0, sem1)
    semis = (semi0, semi1)
    semos = (semo0, semo1)

    def stage_idx(b, k):
        ci = wid + _NW * jnp.minimum(k, nc - 1)
        pltpu.async_copy(gidx_hbm.at[pl.ds(ci * (2 * _C), 2 * _C)],
                         gidx_bufs[b], semis[b])

    def wait_idx(b):
        pltpu.make_async_copy(
            gidx_hbm.at[pl.ds(0, 2 * _C)], gidx_bufs[b], semis[b]).wait()

    def gathers(b):
        pltpu.async_copy(zh_hbm.at[gidx_bufs[b].at[pl.ds(0, _C)]],
                         h_bufs[b], sems[b])
        pltpu.async_copy(zt_hbm.at[gidx_bufs[b].at[pl.ds(_C, _C)]],
                         t_bufs[b], sems[b])

    def wait_g(b):
        pltpu.make_async_copy(
            zh_hbm.at[pl.ds(0, _C)], h_bufs[b], sems[b]).wait()
        pltpu.make_async_copy(
            zt_hbm.at[pl.ds(0, _C)], t_bufs[b], sems[b]).wait()

    def wait_out(b):
        pltpu.make_async_copy(
            out_bufs[b], out_hbm.at[pl.ds(0, _C)], semos[b]).wait()

    def compute(b, k):
        hrow, trow, outv = h_bufs[b], t_bufs[b], out_bufs[b]
        lane17 = lane * 17

        def group(g, carry):
            rvec = relv[pl.ds(k * _C + g * _L, _L)]
            for j in range(_L):
                e = g * _L + j
                r = rvec[j]
                acc = jnp.zeros((_L,), jnp.float32)
                for d in range(D // _L):
                    s = pl.ds(d * _L, _L)
                    acc = acc + hrow[e, s] * trow[e, s] * rtab[r, s]
                plsc.store_scatter(scr, [lane17 + j], acc)
            vec = jnp.zeros((_L,), jnp.float32)
            for i in range(_L):
                vec = vec + plsc.load_gather(scr, [lane + i * 17])
            outv[pl.ds(g * _L, _L)] = 1.0 / (1.0 + jnp.exp(-vec))
            return carry

        lax.fori_loop(0, _NG, group, 0)
        ci = wid + _NW * k
        pltpu.async_copy(outv, out_hbm.at[pl.ds(ci * _C, _C)], semos[b])

    def half(b, k):
        wait_g(b)
        stage_idx(b, k + 2)

        @pl.when(k >= 2)
        def _():
            wait_out(b)
        compute(b, k)
        wait_idx(b)
        gathers(b)

    stage_idx(0, 0)
    stage_idx(1, 1)
    wait_idx(0)
    gathers(0)
    wait_idx(1)
    gathers(1)

    def pair(p, carry):
        half(0, 2 * p)
        half(1, 2 * p + 1)
        return carry

    lax.fori_loop(0, _NPAIR, pair, 0)
    wait_g(0)
    wait_g(1)

    @pl.when(nc > 2 * _NPAIR)
    def _():
        wait_out(0)
        compute(0, nc - 1)
    wait_out(0)
    wait_out(1)


@functools.partial(
    pl.kernel,
    mesh=plsc.VectorSubcoreMesh(core_axis_name="c", subcore_axis_name="s"),
    compiler_params=pltpu.CompilerParams(needs_layout_passes=False),
    out_type=jax.ShapeDtypeStruct((E,), jnp.float32),
    scratch_types=[
        pltpu.VMEM((2 * _C,), jnp.int32),
        pltpu.VMEM((2 * _C,), jnp.int32),
        pltpu.VMEM((_NCP * _C,), jnp.int32),
        pltpu.VMEM((_C, D), jnp.float32),
        pltpu.VMEM((_C, D), jnp.float32),
        pltpu.VMEM((_C, D), jnp.float32),
        pltpu.VMEM((_C, D), jnp.float32),
        pltpu.VMEM((R, D), jnp.float32),
        pltpu.VMEM((_C,), jnp.float32),
        pltpu.VMEM((_C,), jnp.float32),
        pltpu.VMEM((_L * 17,), jnp.float32),
        pltpu.SemaphoreType.DMA,
        pltpu.SemaphoreType.DMA,
        pltpu.SemaphoreType.DMA,
        pltpu.SemaphoreType.DMA,
        pltpu.SemaphoreType.DMA,
        pltpu.SemaphoreType.DMA,
    ],
)
def _sc_edge_score(zh, zt, rr, gidx, relp, out,
                   gidx0, gidx1, relv, h0, h1, t0, t1, rtab, outv0, outv1,
                   scr, sem0, sem1, semi0, semi1, semo0, semo1):
    _sc_body(zh, zt, rr, gidx, relp, out,
             gidx0, gidx1, relv, h0, h1, t0, t1, rtab, outv0, outv1, scr,
             sem0, sem1, semi0, semi1, semo0, semo1)


def _pack_gidx(head, tail):
    cols = jnp.stack([head.reshape(_NCH, _C), tail.reshape(_NCH, _C)],
                     axis=1)
    return cols.reshape(-1)


def _pack_rel(rel):
    rows = rel.reshape(_NCH, _C)
    ids = (jnp.arange(_NW)[:, None]
           + _NW * jnp.arange(_NCP)[None, :])
    return rows[jnp.clip(ids, 0, _NCH - 1)].reshape(-1)


def kernel(z, edge_index, rel_type, emb_rel,
           W_head, b_head, W_tail, b_tail, W_rel, b_rel):
    zh, zt = _transform2(z, W_head, b_head, W_tail, b_tail, 2000)
    rr = _transform(emb_rel, W_rel, b_rel, R)
    gidx = _pack_gidx(edge_index[0], edge_index[1])
    relp = _pack_rel(rel_type)
    return _sc_edge_score(zh, zt, rr, gidx, relp)
